# baseline (speedup 1.0000x reference)
import jax
import jax.numpy as jnp
from jax import lax
from jax.experimental import pallas as pl
from jax.experimental.pallas import tpu as pltpu
from jax.experimental.pallas import tpu_sc as plsc

B = 16384
D = 32
V = 1000000
LANES = 16
NUM_WORKERS = 32
B_PER_W = B // NUM_WORKERS
IDX_CHUNK = 128
N_CHUNKS = B_PER_W // IDX_CHUNK
PANEL = 128
TAIL_START = (V // PANEL) * PANEL
TAIL_W = V - TAIL_START
LAST_PANEL = TAIL_START - PANEL
CHUNK8 = 8


def _sc_body(t_idx_hbm, c_idx_hbm, t_tab_hbm, c_tab_hbm, out_hbm,
             t_idx_v, c_idx_v, buf_a, buf_b, pbuf,
             t_tail, c_tail, out_v, sem_a, sem_b):
    nc = 2
    wid = lax.axis_index("s") * nc + lax.axis_index("c")
    blk = wid * N_CHUNKS

    pltpu.sync_copy(t_idx_hbm.at[pl.ds(blk, N_CHUNKS)], t_idx_v)
    pltpu.sync_copy(c_idx_hbm.at[pl.ds(blk, N_CHUNKS)], c_idx_v)
    pltpu.sync_copy(t_tab_hbm.at[:, pl.ds(TAIL_START, TAIL_W)], t_tail)
    pltpu.sync_copy(c_tab_hbm.at[:, pl.ds(TAIL_START, TAIL_W)], c_tail)

    d_lo = lax.iota(jnp.int32, LANES)
    d_hi = d_lo + jnp.int32(LANES)

    def column(tail_ref, buf_slot, vs):
        col = jnp.minimum(vs & jnp.int32(PANEL - 1), jnp.int32(PANEL - 1))
        tcol = jnp.minimum(
            jnp.maximum(vs - jnp.int32(TAIL_START), jnp.int32(0)),
            jnp.int32(TAIL_W - 1))
        is_tail = jnp.broadcast_to(vs >= jnp.int32(TAIL_START), (LANES,))
        colv = jnp.broadcast_to(col, (LANES,))
        tcolv = jnp.broadcast_to(tcol, (LANES,))
        lo = jnp.where(
            is_tail,
            plsc.load_gather(tail_ref, [d_lo, tcolv]),
            plsc.load_gather(buf_slot, [d_lo, colv]))
        hi = jnp.where(
            is_tail,
            plsc.load_gather(tail_ref, [d_hi, tcolv]),
            plsc.load_gather(buf_slot, [d_hi, colv]))
        return lo, hi

    def panel_start(vs):
        return jnp.minimum(
            lax.shift_right_logical(vs, jnp.int32(7)) * jnp.int32(PANEL),
            jnp.int32(LAST_PANEL))

    H = LANES // 2

    def fire(tab_ref, scalars, buf, dsem):
        return [pltpu.async_copy(
            tab_ref.at[:, pl.ds(pl.multiple_of(panel_start(s), PANEL), PANEL)],
            buf.at[u], dsem) for u, s in enumerate(scalars)]

    def drain(buf, dsem):
        for u in range(H):
            pltpu.make_async_copy(
                t_tab_hbm.at[:, pl.ds(0, PANEL)], buf.at[u], dsem).wait()

    def scal(vec, lane0):
        return [lax.squeeze(lax.slice(vec, (lane0 + u,), (lane0 + u + 1,)),
                            (0,)) for u in range(H)]

    def halfround(h, tvec, cvec, tvec_n):
        t_s = scal(tvec, h * H)
        c_s = scal(cvec, h * H)
        t_next = scal(tvec if h == 0 else tvec_n, H - h * H)
        cps_c = fire(c_tab_hbm, c_s, buf_b, sem_b)
        drain(buf_a, sem_a)
        tc = [column(t_tail, buf_a.at[u], t_s[u]) for u in range(H)]
        fire(t_tab_hbm, t_next, buf_a, sem_a)
        for cp in cps_c:
            cp.wait()
        for u in range(H):
            c_lo, c_hi = column(c_tail, buf_b.at[u], c_s[u])
            pbuf[h * H + u] = tc[u][0] * c_lo + tc[u][1] * c_hi

    def round16(g, _):
        tvec = t_idx_v[g // 8, pl.ds((g % 8) * LANES, LANES)]
        cvec = c_idx_v[g // 8, pl.ds((g % 8) * LANES, LANES)]
        gn = jnp.minimum(g + 1, jnp.int32(B_PER_W // LANES - 1))
        tvec_n = t_idx_v[gn // 8, pl.ds((gn % 8) * LANES, LANES)]
        halfround(0, tvec, cvec, tvec_n)
        halfround(1, tvec, cvec, tvec_n)
        lanes16 = lax.iota(jnp.int32, LANES)
        acc0 = jnp.zeros((LANES,), jnp.float32)
        acc1 = jnp.zeros((LANES,), jnp.float32)
        for j in range(0, LANES, 2):
            acc0 = acc0 + plsc.load_gather(
                pbuf, [lanes16, jnp.full((LANES,), j, jnp.int32)])
            acc1 = acc1 + plsc.load_gather(
                pbuf, [lanes16, jnp.full((LANES,), j + 1, jnp.int32)])
        out_v[pl.ds(g * LANES, LANES)] = acc0 + acc1
        return 0

    tvec0 = t_idx_v[0, pl.ds(0, LANES)]
    fire(t_tab_hbm, scal(tvec0, 0), buf_a, sem_a)
    lax.fori_loop(0, B_PER_W // LANES, round16, 0)
    drain(buf_a, sem_a)

    pltpu.sync_copy(out_v, out_hbm.at[pl.ds(wid * B_PER_W, B_PER_W)])


@jax.jit
def _run(t_idx, c_idx, t_tab, c_tab):
    mesh = plsc.VectorSubcoreMesh(core_axis_name="c", subcore_axis_name="s")
    return pl.kernel(
        _sc_body,
        out_type=jax.ShapeDtypeStruct((B,), jnp.float32),
        mesh=mesh,
        compiler_params=pltpu.CompilerParams(needs_layout_passes=False),
        scratch_types=[
            pltpu.VMEM((N_CHUNKS, IDX_CHUNK), jnp.int32),
            pltpu.VMEM((N_CHUNKS, IDX_CHUNK), jnp.int32),
            pltpu.VMEM((LANES // 2, D, PANEL), jnp.float32),
            pltpu.VMEM((LANES // 2, D, PANEL), jnp.float32),
            pltpu.VMEM((LANES, LANES), jnp.float32),
            pltpu.VMEM((D, TAIL_W), jnp.float32),
            pltpu.VMEM((D, TAIL_W), jnp.float32),
            pltpu.VMEM((B_PER_W,), jnp.float32),
            pltpu.SemaphoreType.DMA,
            pltpu.SemaphoreType.DMA,
        ],
    )(t_idx, c_idx, t_tab, c_tab)


def kernel(t_kmer, c_kmer, label, T_weight, C_weight):
    del label
    t_idx = t_kmer.astype(jnp.int32).reshape(B // IDX_CHUNK, IDX_CHUNK)
    c_idx = c_kmer.astype(jnp.int32).reshape(B // IDX_CHUNK, IDX_CHUNK)
    return _run(t_idx, c_idx, T_weight.T, C_weight.T)


# speedup vs baseline: 1.0004x; 1.0004x over previous
import jax
import jax.numpy as jnp
from jax import lax
from jax.experimental import pallas as pl
from jax.experimental.pallas import tpu as pltpu
from jax.experimental.pallas import tpu_sc as plsc

B = 16384
D = 32
V = 1000000
LANES = 16
NUM_WORKERS = 32
B_PER_W = B // NUM_WORKERS
IDX_CHUNK = 128
N_CHUNKS = B_PER_W // IDX_CHUNK
PANEL = 128
TAIL_START = (V // PANEL) * PANEL
TAIL_W = V - TAIL_START
LAST_PANEL = TAIL_START - PANEL
CHUNK8 = 8


def _sc_body(t_idx_hbm, c_idx_hbm, t_tab_hbm, c_tab_hbm, out_hbm,
             t_idx_v, c_idx_v, buf_a, buf_b0, buf_b1, pbuf,
             t_tail, c_tail, out_v, sem_a, sem_b0, sem_b1):
    nc = 2
    wid = lax.axis_index("s") * nc + lax.axis_index("c")
    blk = wid * N_CHUNKS

    pltpu.sync_copy(t_idx_hbm.at[pl.ds(blk, N_CHUNKS)], t_idx_v)
    pltpu.sync_copy(c_idx_hbm.at[pl.ds(blk, N_CHUNKS)], c_idx_v)
    pltpu.sync_copy(t_tab_hbm.at[:, pl.ds(TAIL_START, TAIL_W)], t_tail)
    pltpu.sync_copy(c_tab_hbm.at[:, pl.ds(TAIL_START, TAIL_W)], c_tail)

    d_lo = lax.iota(jnp.int32, LANES)
    d_hi = d_lo + jnp.int32(LANES)

    def column(tail_ref, buf_slot, vs):
        col = jnp.minimum(vs & jnp.int32(PANEL - 1), jnp.int32(PANEL - 1))
        tcol = jnp.minimum(
            jnp.maximum(vs - jnp.int32(TAIL_START), jnp.int32(0)),
            jnp.int32(TAIL_W - 1))
        is_tail = jnp.broadcast_to(vs >= jnp.int32(TAIL_START), (LANES,))
        colv = jnp.broadcast_to(col, (LANES,))
        tcolv = jnp.broadcast_to(tcol, (LANES,))
        lo = jnp.where(
            is_tail,
            plsc.load_gather(tail_ref, [d_lo, tcolv]),
            plsc.load_gather(buf_slot, [d_lo, colv]))
        hi = jnp.where(
            is_tail,
            plsc.load_gather(tail_ref, [d_hi, tcolv]),
            plsc.load_gather(buf_slot, [d_hi, colv]))
        return lo, hi

    def panel_start(vs):
        return jnp.minimum(
            lax.shift_right_logical(vs, jnp.int32(7)) * jnp.int32(PANEL),
            jnp.int32(LAST_PANEL))

    H = LANES // 2

    def fire(tab_ref, scalars, buf, dsem):
        return [pltpu.async_copy(
            tab_ref.at[:, pl.ds(pl.multiple_of(panel_start(s), PANEL), PANEL)],
            buf.at[u], dsem) for u, s in enumerate(scalars)]

    def drain(buf, dsem):
        for u in range(H):
            pltpu.make_async_copy(
                t_tab_hbm.at[:, pl.ds(0, PANEL)], buf.at[u], dsem).wait()

    def scal(vec, lane0):
        return [lax.squeeze(lax.slice(vec, (lane0 + u,), (lane0 + u + 1,)),
                            (0,)) for u in range(H)]

    def halfround(h, tvec, cvec, tvec_n, cvec_n):
        t_s = scal(tvec, h * H)
        c_s = scal(cvec, h * H)
        t_next = scal(tvec if h == 0 else tvec_n, H - h * H)
        c_next = scal(cvec if h == 0 else cvec_n, H - h * H)
        bufb_cur, semb_cur = (buf_b0, sem_b0) if h == 0 else (buf_b1, sem_b1)
        bufb_nxt, semb_nxt = (buf_b1, sem_b1) if h == 0 else (buf_b0, sem_b0)
        fire(c_tab_hbm, c_next, bufb_nxt, semb_nxt)
        drain(buf_a, sem_a)
        tc = [column(t_tail, buf_a.at[u], t_s[u]) for u in range(H)]
        fire(t_tab_hbm, t_next, buf_a, sem_a)
        drain(bufb_cur, semb_cur)
        for u in range(H):
            c_lo, c_hi = column(c_tail, bufb_cur.at[u], c_s[u])
            pbuf[h * H + u] = tc[u][0] * c_lo + tc[u][1] * c_hi

    def round16(g, _):
        tvec = t_idx_v[g // 8, pl.ds((g % 8) * LANES, LANES)]
        cvec = c_idx_v[g // 8, pl.ds((g % 8) * LANES, LANES)]
        gn = jnp.minimum(g + 1, jnp.int32(B_PER_W // LANES - 1))
        tvec_n = t_idx_v[gn // 8, pl.ds((gn % 8) * LANES, LANES)]
        cvec_n = c_idx_v[gn // 8, pl.ds((gn % 8) * LANES, LANES)]
        halfround(0, tvec, cvec, tvec_n, cvec_n)
        halfround(1, tvec, cvec, tvec_n, cvec_n)
        lanes16 = lax.iota(jnp.int32, LANES)
        acc0 = jnp.zeros((LANES,), jnp.float32)
        acc1 = jnp.zeros((LANES,), jnp.float32)
        for j in range(0, LANES, 2):
            acc0 = acc0 + plsc.load_gather(
                pbuf, [lanes16, jnp.full((LANES,), j, jnp.int32)])
            acc1 = acc1 + plsc.load_gather(
                pbuf, [lanes16, jnp.full((LANES,), j + 1, jnp.int32)])
        out_v[pl.ds(g * LANES, LANES)] = acc0 + acc1
        return 0

    tvec0 = t_idx_v[0, pl.ds(0, LANES)]
    cvec0 = c_idx_v[0, pl.ds(0, LANES)]
    fire(t_tab_hbm, scal(tvec0, 0), buf_a, sem_a)
    fire(c_tab_hbm, scal(cvec0, 0), buf_b0, sem_b0)
    lax.fori_loop(0, B_PER_W // LANES, round16, 0)
    drain(buf_a, sem_a)
    drain(buf_b0, sem_b0)

    pltpu.sync_copy(out_v, out_hbm.at[pl.ds(wid * B_PER_W, B_PER_W)])


@jax.jit
def _run(t_idx, c_idx, t_tab, c_tab):
    mesh = plsc.VectorSubcoreMesh(core_axis_name="c", subcore_axis_name="s")
    return pl.kernel(
        _sc_body,
        out_type=jax.ShapeDtypeStruct((B,), jnp.float32),
        mesh=mesh,
        compiler_params=pltpu.CompilerParams(needs_layout_passes=False),
        scratch_types=[
            pltpu.VMEM((N_CHUNKS, IDX_CHUNK), jnp.int32),
            pltpu.VMEM((N_CHUNKS, IDX_CHUNK), jnp.int32),
            pltpu.VMEM((LANES // 2, D, PANEL), jnp.float32),
            pltpu.VMEM((LANES // 2, D, PANEL), jnp.float32),
            pltpu.VMEM((LANES // 2, D, PANEL), jnp.float32),
            pltpu.VMEM((LANES, LANES), jnp.float32),
            pltpu.VMEM((D, TAIL_W), jnp.float32),
            pltpu.VMEM((D, TAIL_W), jnp.float32),
            pltpu.VMEM((B_PER_W,), jnp.float32),
            pltpu.SemaphoreType.DMA,
            pltpu.SemaphoreType.DMA,
            pltpu.SemaphoreType.DMA,
        ],
    )(t_idx, c_idx, t_tab, c_tab)


def kernel(t_kmer, c_kmer, label, T_weight, C_weight):
    del label
    t_idx = t_kmer.astype(jnp.int32).reshape(B // IDX_CHUNK, IDX_CHUNK)
    c_idx = c_kmer.astype(jnp.int32).reshape(B // IDX_CHUNK, IDX_CHUNK)
    return _run(t_idx, c_idx, T_weight.T, C_weight.T)
